# baseline (device time: 5663132 ns/iter reference)
import jax
import jax.numpy as jnp
from jax import lax
from jax.experimental import pallas as pl
from jax.experimental.pallas import tpu as pltpu

N_DEV = 32


def _ring_tables():
    import distributed_mesh_v7x as dm

    devs = list(dm.get_mesh("i", world_size=N_DEV).devices.flat)
    coords = [tuple(getattr(d, "coords", (d.id,)))[:3] for d in devs]
    pos_by_coord = {c: p for p, c in enumerate(coords)}
    xs = sorted({c[0] for c in coords})
    ys = sorted({c[1] for c in coords})
    zs = sorted({c[2] for c in coords})
    full = {(x, y, z) for x in xs for y in ys for z in zs}
    if len(xs) == 2 and set(coords) == full and len(coords) == N_DEV:
        snake = []
        for yi, y in enumerate(ys):
            zz = zs if yi % 2 == 0 else zs[::-1]
            snake.extend((y, z) for z in zz)
        cycle = [(xs[0], y, z) for (y, z) in snake]
        cycle += [(xs[1], y, z) for (y, z) in reversed(snake)]
        cyc = [pos_by_coord[c] for c in cycle]
    else:
        cyc = list(range(N_DEV))
    import os
    if os.environ.get("IDENTITY_RING"):
        cyc = list(range(N_DEV))
    rix = [0] * N_DEV
    for r, p in enumerate(cyc):
        rix[p] = r
    return cyc, rix


def kernel(Q, K, V):
    b, s_loc, h, d = Q.shape
    hd = h * d
    n_hops = N_DEV - 1
    scale = d ** -0.5

    cyc_tab, rix_tab = _ring_tables()

    Kr = K.reshape(b, s_loc, hd)
    Vr = V.reshape(b, s_loc, hd)

    def body(cyc_ref, rix_ref, k_ref, v_ref, kfull, vfull,
             k_send, k_recv, v_send, v_recv):
        my = lax.axis_index("i")
        my_r = rix_ref[my]
        right = cyc_ref[lax.rem(my_r + 1, N_DEV)]
        left = cyc_ref[lax.rem(my_r + N_DEV - 1, N_DEV)]

        barrier_sem = pltpu.get_barrier_semaphore()
        for nbr in (left, right):
            pl.semaphore_signal(
                barrier_sem, inc=1,
                device_id=(nbr,), device_id_type=pl.DeviceIdType.MESH,
            )
        pl.semaphore_wait(barrier_sem, 2)

        kfull[my] = k_ref[...].astype(jnp.bfloat16)
        vfull[my] = v_ref[...].astype(jnp.bfloat16)

        k_all, v_all = [], []
        k_prev = [None, None]
        v_prev = [None, None]
        for t in range(n_hops):
            k_org = cyc_ref[lax.rem(my_r - t + N_DEV, N_DEV)]
            v_org = cyc_ref[lax.rem(my_r + t, N_DEV)]
            for sub in range(2):
                if t > 0:
                    k_prev[sub].wait_recv()
                k_rdma = pltpu.make_async_remote_copy(
                    src_ref=kfull.at[k_org, sub],
                    dst_ref=kfull.at[k_org, sub],
                    send_sem=k_send.at[t, sub],
                    recv_sem=k_recv.at[t, sub],
                    device_id=(right,),
                    device_id_type=pl.DeviceIdType.MESH,
                )
                k_rdma.start()
                k_prev[sub] = k_rdma
                k_all.append(k_rdma)
                if t > 0:
                    v_prev[sub].wait_recv()
                v_rdma = pltpu.make_async_remote_copy(
                    src_ref=vfull.at[v_org, sub],
                    dst_ref=vfull.at[v_org, sub],
                    send_sem=v_send.at[t, sub],
                    recv_sem=v_recv.at[t, sub],
                    device_id=(left,),
                    device_id_type=pl.DeviceIdType.MESH,
                )
                v_rdma.start()
                v_prev[sub] = v_rdma
                v_all.append(v_rdma)
        for sub in range(2):
            k_prev[sub].wait_recv()
            v_prev[sub].wait_recv()

        for rdma in k_all:
            rdma.wait_send()
        for rdma in v_all:
            rdma.wait_send()

    kg, vg = pl.pallas_call(
        body,
        out_shape=[
            jax.ShapeDtypeStruct((N_DEV, b, s_loc, hd), jnp.bfloat16),
            jax.ShapeDtypeStruct((N_DEV, b, s_loc, hd), jnp.bfloat16),
        ],
        in_specs=[pl.BlockSpec(memory_space=pltpu.SMEM)] * 2
        + [pl.BlockSpec(memory_space=pltpu.VMEM)] * 2,
        out_specs=[pl.BlockSpec(memory_space=pltpu.VMEM)] * 2,
        scratch_shapes=[
            pltpu.SemaphoreType.DMA((n_hops, 2)),
            pltpu.SemaphoreType.DMA((n_hops, 2)),
            pltpu.SemaphoreType.DMA((n_hops, 2)),
            pltpu.SemaphoreType.DMA((n_hops, 2)),
        ],
        compiler_params=pltpu.CompilerParams(
            collective_id=0, vmem_limit_bytes=100 * 1024 * 1024,
        ),
    )(
        jnp.array(cyc_tab, dtype=jnp.int32),
        jnp.array(rix_tab, dtype=jnp.int32),
        Kr, Vr,
    )

    Kb = kg.transpose(1, 0, 2, 3).reshape(b, N_DEV * s_loc, h, d)
    Vb = vg.transpose(1, 0, 2, 3).reshape(b, N_DEV * s_loc, h, d)
    Qb = Q.astype(jnp.bfloat16)
    S = jnp.einsum(
        "bqhd,bkhd->bhqk", Qb, Kb, preferred_element_type=jnp.float32,
    ) * scale
    m = S.max(-1, keepdims=True)
    P = jnp.exp(S - m)
    P = P / P.sum(-1, keepdims=True)
    return jnp.einsum(
        "bhqk,bkhd->bqhd", P.astype(jnp.bfloat16), Vb,
        preferred_element_type=jnp.float32,
    )


# device time: 282324 ns/iter; 20.0590x vs baseline; 20.0590x over previous
import jax
import jax.numpy as jnp
from jax import lax
from jax.experimental import pallas as pl
from jax.experimental.pallas import tpu as pltpu

N_DEV = 32


def _ring_tables():
    import distributed_mesh_v7x as dm

    devs = list(dm.get_mesh("i", world_size=N_DEV).devices.flat)
    coords = [tuple(getattr(d, "coords", (d.id,)))[:3] for d in devs]
    pos_by_coord = {c: p for p, c in enumerate(coords)}
    xs = sorted({c[0] for c in coords})
    ys = sorted({c[1] for c in coords})
    zs = sorted({c[2] for c in coords})
    full = {(x, y, z) for x in xs for y in ys for z in zs}
    if len(xs) == 2 and set(coords) == full and len(coords) == N_DEV:
        snake = []
        for yi, y in enumerate(ys):
            zz = zs if yi % 2 == 0 else zs[::-1]
            snake.extend((y, z) for z in zz)
        cycle = [(xs[0], y, z) for (y, z) in snake]
        cycle += [(xs[1], y, z) for (y, z) in reversed(snake)]
        cyc = [pos_by_coord[c] for c in cycle]
    else:
        cyc = list(range(N_DEV))
    import os
    if os.environ.get("IDENTITY_RING"):
        cyc = list(range(N_DEV))
    rix = [0] * N_DEV
    for r, p in enumerate(cyc):
        rix[p] = r
    return cyc, rix


def kernel(Q, K, V):
    b, s_loc, h, d = Q.shape
    hd = h * d
    n_hops = N_DEV - 1
    scale = d ** -0.5

    cyc_tab, rix_tab = _ring_tables()

    Kr = K.reshape(b, s_loc, hd)
    Vr = V.reshape(b, s_loc, hd)

    def body(cyc_ref, rix_ref, k_ref, v_ref, kfull, vfull,
             k_send, k_recv, v_send, v_recv):
        my = lax.axis_index("i")
        my_r = rix_ref[my]
        right = cyc_ref[lax.rem(my_r + 1, N_DEV)]
        left = cyc_ref[lax.rem(my_r + N_DEV - 1, N_DEV)]

        barrier_sem = pltpu.get_barrier_semaphore()
        for nbr in (left, right):
            pl.semaphore_signal(
                barrier_sem, inc=1,
                device_id=(nbr,), device_id_type=pl.DeviceIdType.MESH,
            )
        pl.semaphore_wait(barrier_sem, 2)

        kfull[my] = k_ref[...].astype(jnp.bfloat16)
        vfull[my] = v_ref[...].astype(jnp.bfloat16)

        k_all, v_all = [], []
        k_prev = [None, None]
        v_prev = [None, None]
        for t in range(n_hops):
            k_org = cyc_ref[lax.rem(my_r - t + N_DEV, N_DEV)]
            v_org = cyc_ref[lax.rem(my_r + t, N_DEV)]
            for sub in range(2):
                if t > 0:
                    k_prev[sub].wait_recv()
                k_rdma = pltpu.make_async_remote_copy(
                    src_ref=kfull.at[k_org, sub],
                    dst_ref=kfull.at[k_org, sub],
                    send_sem=k_send.at[t, sub],
                    recv_sem=k_recv.at[t, sub],
                    device_id=(right,),
                    device_id_type=pl.DeviceIdType.MESH,
                )
                k_rdma.start()
                k_prev[sub] = k_rdma
                k_all.append(k_rdma)
                if t > 0:
                    v_prev[sub].wait_recv()
                v_rdma = pltpu.make_async_remote_copy(
                    src_ref=vfull.at[v_org, sub],
                    dst_ref=vfull.at[v_org, sub],
                    send_sem=v_send.at[t, sub],
                    recv_sem=v_recv.at[t, sub],
                    device_id=(left,),
                    device_id_type=pl.DeviceIdType.MESH,
                )
                v_rdma.start()
                v_prev[sub] = v_rdma
                v_all.append(v_rdma)
        for sub in range(2):
            k_prev[sub].wait_recv()
            v_prev[sub].wait_recv()

        for rdma in k_all:
            rdma.wait_send()
        for rdma in v_all:
            rdma.wait_send()

    kg, vg = pl.pallas_call(
        body,
        out_shape=[
            jax.ShapeDtypeStruct((N_DEV, b, s_loc, hd), jnp.bfloat16),
            jax.ShapeDtypeStruct((N_DEV, b, s_loc, hd), jnp.bfloat16),
        ],
        in_specs=[pl.BlockSpec(memory_space=pltpu.SMEM)] * 2
        + [pl.BlockSpec(memory_space=pltpu.VMEM)] * 2,
        out_specs=[pl.BlockSpec(memory_space=pltpu.VMEM)] * 2,
        scratch_shapes=[
            pltpu.SemaphoreType.DMA((n_hops, 2)),
            pltpu.SemaphoreType.DMA((n_hops, 2)),
            pltpu.SemaphoreType.DMA((n_hops, 2)),
            pltpu.SemaphoreType.DMA((n_hops, 2)),
        ],
        compiler_params=pltpu.CompilerParams(
            collective_id=0, vmem_limit_bytes=100 * 1024 * 1024,
        ),
    )(
        jnp.array(cyc_tab, dtype=jnp.int32),
        jnp.array(rix_tab, dtype=jnp.int32),
        Kr, Vr,
    )

    def attn_body(q_ref, kg_ref, vg_ref, out_ref):
        for bb in range(b):
            for hh in range(h):
                q = q_ref[bb, :, hh * d:(hh + 1) * d].astype(jnp.bfloat16)
                k = kg_ref[:, bb, :, hh * d:(hh + 1) * d].reshape(
                    N_DEV * s_loc, d)
                s = lax.dot_general(
                    q, k, (((1,), (1,)), ((), ())),
                    preferred_element_type=jnp.float32,
                ) * scale
                m = jnp.max(s, axis=-1, keepdims=True)
                p = jnp.exp(s - m)
                l = jnp.sum(p, axis=-1, keepdims=True)
                p = (p / l).astype(jnp.bfloat16)
                v = vg_ref[:, bb, :, hh * d:(hh + 1) * d].reshape(
                    N_DEV * s_loc, d)
                o = lax.dot_general(
                    p, v, (((1,), (0,)), ((), ())),
                    preferred_element_type=jnp.float32,
                )
                out_ref[bb, :, hh * d:(hh + 1) * d] = o

    out = pl.pallas_call(
        attn_body,
        out_shape=jax.ShapeDtypeStruct((b, s_loc, hd), jnp.float32),
        in_specs=[pl.BlockSpec(memory_space=pltpu.VMEM)] * 3,
        out_specs=pl.BlockSpec(memory_space=pltpu.VMEM),
        compiler_params=pltpu.CompilerParams(
            vmem_limit_bytes=60 * 1024 * 1024,
        ),
    )(Q.reshape(b, s_loc, hd), kg, vg)
    return out.reshape(b, s_loc, h, d)


# device time: 258401 ns/iter; 21.9161x vs baseline; 1.0926x over previous
import jax
import jax.numpy as jnp
from jax import lax
from jax.experimental import pallas as pl
from jax.experimental.pallas import tpu as pltpu

N_DEV = 32


def _ring_tables():
    import distributed_mesh_v7x as dm

    devs = list(dm.get_mesh("i", world_size=N_DEV).devices.flat)
    coords = [tuple(getattr(d, "coords", (d.id,)))[:3] for d in devs]
    pos_by_coord = {c: p for p, c in enumerate(coords)}
    xs = sorted({c[0] for c in coords})
    ys = sorted({c[1] for c in coords})
    zs = sorted({c[2] for c in coords})
    full = {(x, y, z) for x in xs for y in ys for z in zs}
    if len(xs) == 2 and set(coords) == full and len(coords) == N_DEV:
        snake = []
        for yi, y in enumerate(ys):
            zz = zs if yi % 2 == 0 else zs[::-1]
            snake.extend((y, z) for z in zz)
        cycle = [(xs[0], y, z) for (y, z) in snake]
        cycle += [(xs[1], y, z) for (y, z) in reversed(snake)]
        cyc = [pos_by_coord[c] for c in cycle]
    else:
        cyc = list(range(N_DEV))
    import os
    if os.environ.get("IDENTITY_RING"):
        cyc = list(range(N_DEV))
    rix = [0] * N_DEV
    for r, p in enumerate(cyc):
        rix[p] = r
    return cyc, rix


def kernel(Q, K, V):
    b, s_loc, h, d = Q.shape
    hd = h * d
    n_hops = N_DEV - 1
    scale = d ** -0.5

    cyc_tab, rix_tab = _ring_tables()

    Qr = Q.reshape(b, s_loc, hd)
    Kr = K.reshape(b, s_loc, hd)
    Vr = V.reshape(b, s_loc, hd)

    def body(cyc_ref, rix_ref, q_ref, k_ref, v_ref, out_ref, kfull, vfull,
             k_send, k_recv, v_send, v_recv):
        my = lax.axis_index("i")
        my_r = rix_ref[my]
        right = cyc_ref[lax.rem(my_r + 1, N_DEV)]
        left = cyc_ref[lax.rem(my_r + N_DEV - 1, N_DEV)]

        barrier_sem = pltpu.get_barrier_semaphore()
        for nbr in (left, right):
            pl.semaphore_signal(
                barrier_sem, inc=1,
                device_id=(nbr,), device_id_type=pl.DeviceIdType.MESH,
            )
        pl.semaphore_wait(barrier_sem, 2)

        kfull[my] = k_ref[...].astype(jnp.bfloat16)
        vfull[my] = v_ref[...].astype(jnp.bfloat16)

        k_all, v_all = [], []
        k_prev = [None, None]
        v_prev = [None, None]
        for t in range(n_hops):
            k_org = cyc_ref[lax.rem(my_r - t + N_DEV, N_DEV)]
            v_org = cyc_ref[lax.rem(my_r + t, N_DEV)]
            for sub in range(2):
                if t > 0:
                    k_prev[sub].wait_recv()
                k_rdma = pltpu.make_async_remote_copy(
                    src_ref=kfull.at[k_org, sub],
                    dst_ref=kfull.at[k_org, sub],
                    send_sem=k_send.at[t, sub],
                    recv_sem=k_recv.at[t, sub],
                    device_id=(right,),
                    device_id_type=pl.DeviceIdType.MESH,
                )
                k_rdma.start()
                k_prev[sub] = k_rdma
                k_all.append(k_rdma)
                if t > 0:
                    v_prev[sub].wait_recv()
                v_rdma = pltpu.make_async_remote_copy(
                    src_ref=vfull.at[v_org, sub],
                    dst_ref=vfull.at[v_org, sub],
                    send_sem=v_send.at[t, sub],
                    recv_sem=v_recv.at[t, sub],
                    device_id=(left,),
                    device_id_type=pl.DeviceIdType.MESH,
                )
                v_rdma.start()
                v_prev[sub] = v_rdma
                v_all.append(v_rdma)
        for sub in range(2):
            k_prev[sub].wait_recv()
            v_prev[sub].wait_recv()

        for rdma in k_all:
            rdma.wait_send()
        for rdma in v_all:
            rdma.wait_send()

        for bb in range(b):
            for hh in range(h):
                q = q_ref[bb, :, hh * d:(hh + 1) * d].astype(jnp.bfloat16)
                k = kfull[:, bb, :, hh * d:(hh + 1) * d].reshape(
                    N_DEV * s_loc, d)
                s = lax.dot_general(
                    q, k, (((1,), (1,)), ((), ())),
                    preferred_element_type=jnp.float32,
                ) * scale
                m = jnp.max(s, axis=-1, keepdims=True)
                p = jnp.exp(s - m)
                l = jnp.sum(p, axis=-1, keepdims=True)
                p = (p / l).astype(jnp.bfloat16)
                v = vfull[:, bb, :, hh * d:(hh + 1) * d].reshape(
                    N_DEV * s_loc, d)
                o = lax.dot_general(
                    p, v, (((1,), (0,)), ((), ())),
                    preferred_element_type=jnp.float32,
                )
                out_ref[bb, :, hh * d:(hh + 1) * d] = o

    out = pl.pallas_call(
        body,
        out_shape=jax.ShapeDtypeStruct((b, s_loc, hd), jnp.float32),
        in_specs=[pl.BlockSpec(memory_space=pltpu.SMEM)] * 2
        + [pl.BlockSpec(memory_space=pltpu.VMEM)] * 3,
        out_specs=pl.BlockSpec(memory_space=pltpu.VMEM),
        scratch_shapes=[
            pltpu.VMEM((N_DEV, b, s_loc, hd), jnp.bfloat16),
            pltpu.VMEM((N_DEV, b, s_loc, hd), jnp.bfloat16),
            pltpu.SemaphoreType.DMA((n_hops, 2)),
            pltpu.SemaphoreType.DMA((n_hops, 2)),
            pltpu.SemaphoreType.DMA((n_hops, 2)),
            pltpu.SemaphoreType.DMA((n_hops, 2)),
        ],
        compiler_params=pltpu.CompilerParams(
            collective_id=0, vmem_limit_bytes=100 * 1024 * 1024,
        ),
    )(
        jnp.array(cyc_tab, dtype=jnp.int32),
        jnp.array(rix_tab, dtype=jnp.int32),
        Qr, Kr, Vr,
    )
    return out.reshape(b, s_loc, h, d)
